# Initial kernel scaffold; baseline (speedup 1.0000x reference)
import jax
import jax.numpy as jnp
from jax.experimental import pallas as pl


def kernel(node_ids, agg_messages, timestamps, memory, last_update, W_ih, W_hh, b_ih, b_hh):
    raise NotImplementedError("write your pallas kernel here")


# speedup vs baseline: 2.2036x; 2.2036x over previous
import functools

import jax
import jax.numpy as jnp
from jax import lax
from jax.experimental import pallas as pl
from jax.experimental.pallas import tpu as pltpu
from jax.experimental.pallas import tpu_sc as plsc

N_NODES = 100000
D = 512
M = 16384
NC = 2
NS = 16
L = 16
NW = NC * NS
RANGE = 3200
NPAD = NW * RANGE
ROWS_PER_W = M // NW
GCHUNK = 64
K = 64
MAXW = min(RANGE, M)
NCH = MAXW // K
IDC = M // L
SLOTP = RANGE

_mesh = plsc.VectorSubcoreMesh(core_axis_name="c", subcore_axis_name="s")


def _wid():
    return lax.axis_index("s") * NC + lax.axis_index("c")


@functools.partial(
    pl.kernel,
    out_type=jax.ShapeDtypeStruct((M, D), jnp.float32),
    mesh=_mesh,
    scratch_types=[
        pltpu.VMEM((GCHUNK,), jnp.int32),
        pltpu.VMEM((GCHUNK,), jnp.int32),
        pltpu.VMEM((GCHUNK, D), jnp.float32),
        pltpu.VMEM((GCHUNK, D), jnp.float32),
        pltpu.SemaphoreType.DMA,
        pltpu.SemaphoreType.DMA,
    ],
)
def _gather_rows(mem_hbm, ids_hbm, out_hbm, idx_v, idx2_v, rows_v, rows2_v,
                 sem, semb):
    base = _wid() * ROWS_PER_W

    def body(c, carry):
        off0 = base + (2 * c) * GCHUNK
        off1 = off0 + GCHUNK
        pltpu.sync_copy(ids_hbm.at[pl.ds(off0, GCHUNK)], idx_v)
        pltpu.sync_copy(ids_hbm.at[pl.ds(off1, GCHUNK)], idx2_v)
        g0 = pltpu.async_copy(mem_hbm.at[idx_v], rows_v, sem)
        g1 = pltpu.async_copy(mem_hbm.at[idx2_v], rows2_v, semb)
        g0.wait()
        s0 = pltpu.async_copy(rows_v, out_hbm.at[pl.ds(off0, GCHUNK)], sem)
        g1.wait()
        s1 = pltpu.async_copy(rows2_v, out_hbm.at[pl.ds(off1, GCHUNK)], semb)
        s0.wait()
        s1.wait()
        return carry

    lax.fori_loop(0, ROWS_PER_W // (2 * GCHUNK), body, 0)


BM = 1024


def _gru_body(x_ref, h_ref, wih_ref, whh_ref, bih_ref, bhh_ref, out_ref):
    x = x_ref[...]
    h = h_ref[...]
    dn = (((1,), (1,)), ((), ()))
    gi = lax.dot_general(x, wih_ref[...], dn,
                         preferred_element_type=jnp.float32) + bih_ref[...]
    gh = lax.dot_general(h, whh_ref[...], dn,
                         preferred_element_type=jnp.float32) + bhh_ref[...]
    r = jax.nn.sigmoid(gi[:, :D] + gh[:, :D])
    z = jax.nn.sigmoid(gi[:, D:2 * D] + gh[:, D:2 * D])
    n = jnp.tanh(gi[:, 2 * D:] + r * gh[:, 2 * D:])
    out_ref[...] = (1.0 - z) * n + z * h


_gru = pl.pallas_call(
    _gru_body,
    grid=(M // BM,),
    in_specs=[
        pl.BlockSpec((BM, D), lambda i: (i, 0)),
        pl.BlockSpec((BM, D), lambda i: (i, 0)),
        pl.BlockSpec((3 * D, D), lambda i: (0, 0)),
        pl.BlockSpec((3 * D, D), lambda i: (0, 0)),
        pl.BlockSpec((1, 3 * D), lambda i: (0, 0)),
        pl.BlockSpec((1, 3 * D), lambda i: (0, 0)),
    ],
    out_specs=pl.BlockSpec((BM, D), lambda i: (i, 0)),
    out_shape=jax.ShapeDtypeStruct((M, D), jnp.float32),
)


@functools.partial(
    pl.kernel,
    mesh=_mesh,
    compiler_params=pltpu.CompilerParams(needs_layout_passes=False),
    scratch_types=[
        pltpu.VMEM((M,), jnp.int32),
        pltpu.VMEM((SLOTP,), jnp.int32),
        pltpu.VMEM((NCH, K), jnp.int32),
        pltpu.VMEM((NCH, K), jnp.int32),
        pltpu.VMEM((K, D), jnp.float32),
        pltpu.VMEM((K, D), jnp.float32),
        pltpu.VMEM((M,), jnp.float32),
        pltpu.VMEM((RANGE,), jnp.float32),
        pltpu.SemaphoreType.DMA,
        pltpu.SemaphoreType.DMA,
    ],
)
def _scatter_rows(ids_hbm, newmem_hbm, ts_hbm, mem_hbm, lu_hbm,
                  ids_v, slot_v, plist_v, nlist_v, rows_v, rows2_v,
                  ts_v, lu_local, sem, semb):
    lo = _wid() * RANGE
    lane = lax.iota(jnp.int32, L)

    pltpu.sync_copy(ids_hbm, ids_v)
    pltpu.sync_copy(ts_hbm, ts_v)
    pltpu.sync_copy(lu_hbm.at[pl.ds(lo, RANGE)], lu_local)

    def init_b(c, carry):
        slot_v[pl.ds(c * L, L)] = jnp.full((L,), -1, jnp.int32)
        return carry

    lax.fori_loop(0, SLOTP // L, init_b, 0)

    nlane = jnp.minimum(lane + 1, L - 1)

    def p1(c, carry):
        for u in range(4):
            cc = c * 4 + u
            idx = ids_v[pl.ds(cc * L, L)]
            keym = idx * L + lane
            pos = cc * L + lane
            sk, sp = plsc.sort_key_val(keym, pos)
            sidx = lax.shift_right_logical(sk, 4)
            nxt = lax.gather(
                sk, nlane[:, None],
                lax.GatherDimensionNumbers(offset_dims=(),
                                           collapsed_slice_dims=(0,),
                                           start_index_map=(0,)),
                slice_sizes=(1,),
                mode=lax.GatherScatterMode.PROMISE_IN_BOUNDS)
            islast = (sidx != lax.shift_right_logical(nxt, 4)) | (lane == L - 1)
            mine = (sidx >= lo) & (sidx < lo + RANGE)
            m = islast & mine
            loc = jnp.where(m, sidx - lo, 0)
            plsc.store_scatter(slot_v, [loc], sp, mask=m)
        return carry

    lax.fori_loop(0, IDC // 4, p1, 0)

    def p2(c, cursor):
        sv = slot_v[pl.ds(c * L, L)]
        m = sv >= 0
        mi = m.astype(jnp.int32)
        offs = cursor + plsc.cumsum(mi) - 1
        o = jnp.where(m, offs, 0)
        node = lo + c * L + lane
        plsc.store_scatter(plist_v, [o // K, o % K], sv, mask=m)
        plsc.store_scatter(nlist_v, [o // K, o % K], node, mask=m)
        tsv = plsc.load_gather(ts_v, [jnp.where(m, sv, 0)])
        cur = lu_local[pl.ds(c * L, L)]
        lu_local[pl.ds(c * L, L)] = jnp.where(m, tsv, cur)
        return cursor + jnp.sum(mi)

    count = lax.fori_loop(0, SLOTP // L, p2, jnp.int32(0))

    padded = ((count + 2 * K - 1) // (2 * K)) * (2 * K)
    lastidx = jnp.maximum(count - 1, 0)
    li0 = jnp.broadcast_to(lastidx // K, (L,))
    li1 = jnp.broadcast_to(lastidx % K, (L,))
    lastp = plsc.load_gather(plist_v, [li0, li1])
    lastn = plsc.load_gather(nlist_v, [li0, li1])

    def pad_b(k, carry):
        offs = count + k * L + lane
        m = offs < padded
        o = jnp.where(m, offs, 0)
        plsc.store_scatter(plist_v, [o // K, o % K], lastp, mask=m)
        plsc.store_scatter(nlist_v, [o // K, o % K], lastn, mask=m)
        return carry

    lax.fori_loop(0, 2 * K // L, pad_b, 0)

    pltpu.sync_copy(lu_local, lu_hbm.at[pl.ds(lo, RANGE)])

    def cp2(jj, carry):
        j0 = 2 * jj
        g0 = pltpu.async_copy(newmem_hbm.at[plist_v.at[j0]], rows_v, sem)
        g1 = pltpu.async_copy(newmem_hbm.at[plist_v.at[j0 + 1]], rows2_v, semb)
        g0.wait()
        s0 = pltpu.async_copy(rows_v, mem_hbm.at[nlist_v.at[j0]], sem)
        g1.wait()
        s1 = pltpu.async_copy(rows2_v, mem_hbm.at[nlist_v.at[j0 + 1]], semb)
        s0.wait()
        s1.wait()
        return carry

    lax.fori_loop(0, padded // (2 * K), cp2, 0)


def kernel(node_ids, agg_messages, timestamps, memory, last_update,
           W_ih, W_hh, b_ih, b_hh):
    ids = node_ids.astype(jnp.int32)
    mem_ref = jax.new_ref(memory)
    lu_pad = jnp.concatenate(
        [last_update, jnp.zeros((NPAD - N_NODES,), jnp.float32)])
    lu_ref = jax.new_ref(lu_pad)
    cur_mem = _gather_rows(memory, ids)
    new_mem = _gru(agg_messages, cur_mem, W_ih, W_hh,
                   b_ih.reshape(1, 3 * D), b_hh.reshape(1, 3 * D))
    _scatter_rows(ids, new_mem, timestamps, mem_ref, lu_ref)
    return mem_ref[...], lu_ref[...][:N_NODES], new_mem
